# baseline (speedup 1.0000x reference)
import functools

import jax
import jax.numpy as jnp
from jax import lax
from jax.experimental import pallas as pl
from jax.experimental.pallas import tpu as pltpu
from jax.experimental.pallas import tpu_sc as plsc

DIM = 32
LANES = 16
NC = 2
NS = 16
NW = NC * NS

BATCH = 16384
NEG = 64
B_PER_W = BATCH // NW
NEGS_PER_W = B_PER_W * NEG
STREAM = 128
CHUNK = 512
N_CHUNK = NEGS_PER_W // CHUNK
GROUPS = CHUNK // LANES
IDX_ROWS_PER_W = NEGS_PER_W // STREAM
HIDX_ROWS = B_PER_W // STREAM
N_ENT_ROWS = 1000000


_TPITCH = LANES + 1


def _lane_sums(ps, tbuf):
    iota = lax.iota(jnp.int32, LANES)
    iota_p = iota * _TPITCH
    for j, p in enumerate(ps):
        plsc.store_scatter(tbuf, [iota_p + j], p)
    vs = [plsc.load_gather(tbuf, [iota + (_TPITCH * l)])
          for l in range(LANES)]
    while len(vs) > 1:
        vs = [vs[i] + vs[i + 1] for i in range(0, len(vs), 2)]
    return vs[0]


def _sc_body(h_hbm, r_hbm, t_hbm, neg_hbm, ent_flat_hbm, rel_hbm,
             pos_out, neg_out,
             hidx_v, ridx_v, tidx_v, negidx_v, buf_hr, buf_rows,
             buf_rows1, posstage, outstage, outstage1, tbuf, tbuf1,
             sem, semg0, semg1, semo0, semo1):
    ent_hbm = ent_flat_hbm
    wid = lax.axis_index("s") * NC + lax.axis_index("c")
    iota = lax.iota(jnp.int32, LANES)

    pltpu.sync_copy(h_hbm.at[pl.ds(wid * HIDX_ROWS, HIDX_ROWS)], hidx_v)
    pltpu.sync_copy(r_hbm.at[pl.ds(wid * HIDX_ROWS, HIDX_ROWS)], ridx_v)
    pltpu.sync_copy(t_hbm.at[pl.ds(wid * HIDX_ROWS, HIDX_ROWS)], tidx_v)
    pltpu.sync_copy(neg_hbm.at[pl.ds(wid * IDX_ROWS_PER_W, IDX_ROWS_PER_W)],
                    negidx_v)

    cps = [pltpu.async_copy(ent_hbm.at[hidx_v.at[j]],
                            buf_hr.at[pl.ds(j * STREAM, STREAM)], sem)
           for j in range(HIDX_ROWS)]
    for cp in cps:
        cp.wait()
    cps = [pltpu.async_copy(rel_hbm.at[ridx_v.at[j]],
                            buf_rows.at[pl.ds(j * STREAM, STREAM)], sem)
           for j in range(HIDX_ROWS)]
    for cp in cps:
        cp.wait()

    def hr_body(i, _):
        buf_hr[i, pl.ds(0, LANES)] = (buf_hr[i, pl.ds(0, LANES)]
                                      + buf_rows[i, pl.ds(0, LANES)])
        buf_hr[i, pl.ds(LANES, LANES)] = (buf_hr[i, pl.ds(LANES, LANES)]
                                          + buf_rows[i, pl.ds(LANES, LANES)])
        return 0
    lax.fori_loop(0, B_PER_W, hr_body, 0)

    cps = [pltpu.async_copy(ent_hbm.at[tidx_v.at[j]],
                            buf_rows.at[pl.ds(j * STREAM, STREAM)], sem)
           for j in range(HIDX_ROWS)]
    for cp in cps:
        cp.wait()

    def pos_body(g, _):
        base = g * LANES
        ps = []
        for j in range(LANES):
            row = base + j
            d0 = buf_hr[row, pl.ds(0, LANES)] - buf_rows[row, pl.ds(0, LANES)]
            d1 = (buf_hr[row, pl.ds(LANES, LANES)]
                  - buf_rows[row, pl.ds(LANES, LANES)])
            ps.append(d0 * d0 + d1 * d1)
        posstage[pl.ds(base, LANES)] = _lane_sums(ps, tbuf)
        return 0
    lax.fori_loop(0, B_PER_W // LANES, pos_body, 0)
    pltpu.sync_copy(posstage, pos_out.at[pl.ds(wid * B_PER_W, B_PER_W)])

    def fire(c, buf, semg):
        for j in range(4):
            pltpu.async_copy(ent_hbm.at[negidx_v.at[c * 4 + j]],
                             buf.at[pl.ds(j * STREAM, STREAM)], semg)

    def drain(buf, semg):
        for j in range(4):
            pltpu.make_async_copy(ent_hbm.at[negidx_v.at[j]],
                                  buf.at[pl.ds(j * STREAM, STREAM)],
                                  semg).wait()

    def fire_out(c, ostage, semo):
        pltpu.async_copy(ostage,
                         neg_out.at[pl.ds(wid * NEGS_PER_W + c * CHUNK,
                                          CHUNK)], semo)

    def drain_out(ostage, semo):
        pltpu.make_async_copy(ostage, neg_out.at[pl.ds(wid * NEGS_PER_W,
                                                       CHUNK)], semo).wait()

    def compute(c, buf, ostage):
        def row_body(b, _):
            local = c * 8 + b
            hr0 = buf_hr[local, pl.ds(0, LANES)]
            hr1 = buf_hr[local, pl.ds(LANES, LANES)]
            for g in range(NEG // LANES):
                base = b * NEG + g * LANES
                ps = []
                for j in range(LANES):
                    row = base + j
                    x0 = buf[row, pl.ds(0, LANES)]
                    x1 = buf[row, pl.ds(LANES, LANES)]
                    d0 = x0 - hr0
                    d1 = x1 - hr1
                    ps.append(d0 * d0 + d1 * d1)
                ostage[pl.ds(base, LANES)] = _lane_sums(
                    ps, tbuf if g % 2 == 0 else tbuf1)
            return 0
        lax.fori_loop(0, CHUNK // NEG, row_body, 0)

    fire(0, buf_rows, semg0)

    def pair_body(p, _):
        c0 = 2 * p
        fire(c0 + 1, buf_rows1, semg1)
        drain(buf_rows, semg0)

        @pl.when(p > 0)
        def _():
            drain_out(outstage, semo0)
        compute(c0, buf_rows, outstage)
        fire_out(c0, outstage, semo0)

        @pl.when(p + 1 < N_CHUNK // 2)
        def _():
            fire(c0 + 2, buf_rows, semg0)
        drain(buf_rows1, semg1)

        @pl.when(p > 0)
        def _():
            drain_out(outstage1, semo1)
        compute(c0 + 1, buf_rows1, outstage1)
        fire_out(c0 + 1, outstage1, semo1)
        return 0
    lax.fori_loop(0, N_CHUNK // 2, pair_body, 0)
    drain_out(outstage, semo0)
    drain_out(outstage1, semo1)


_sc_call = pl.kernel(
    _sc_body,
    out_type=[jax.ShapeDtypeStruct((BATCH,), jnp.float32),
              jax.ShapeDtypeStruct((BATCH * NEG,), jnp.float32)],
    mesh=plsc.VectorSubcoreMesh(core_axis_name="c", subcore_axis_name="s",
                                num_cores=NC, num_subcores=NS),
    compiler_params=pltpu.CompilerParams(needs_layout_passes=False,
                                         use_tc_tiling_on_sc=False),
    scratch_types=[
        pltpu.VMEM((HIDX_ROWS, STREAM), jnp.int32),
        pltpu.VMEM((HIDX_ROWS, STREAM), jnp.int32),
        pltpu.VMEM((HIDX_ROWS, STREAM), jnp.int32),
        pltpu.VMEM((IDX_ROWS_PER_W, STREAM), jnp.int32),
        pltpu.VMEM((CHUNK, DIM), jnp.float32),
        pltpu.VMEM((CHUNK, DIM), jnp.float32),
        pltpu.VMEM((CHUNK, DIM), jnp.float32),
        pltpu.VMEM((B_PER_W,), jnp.float32),
        pltpu.VMEM((CHUNK,), jnp.float32),
        pltpu.VMEM((CHUNK,), jnp.float32),
        pltpu.VMEM((LANES * _TPITCH,), jnp.float32),
        pltpu.VMEM((LANES * _TPITCH,), jnp.float32),
        pltpu.SemaphoreType.DMA,
        pltpu.SemaphoreType.DMA,
        pltpu.SemaphoreType.DMA,
        pltpu.SemaphoreType.DMA,
        pltpu.SemaphoreType.DMA,
    ],
)


_TRB = 8192


def _tc_relayout_body(src_ref, dst_ref, scratch):
    scratch[...] = src_ref[...].T
    for a in range(4):
        dst_ref[:, a * DIM:(a + 1) * DIM] = scratch[
            pl.Slice(a, _TRB // 4, 4), :]


_tc_relayout = pl.pallas_call(
    _tc_relayout_body,
    grid=(123,),
    in_specs=[pl.BlockSpec((DIM, _TRB), lambda j: (0, j))],
    out_specs=pl.BlockSpec((_TRB // 4, 128), lambda j: (j, 0)),
    out_shape=jax.ShapeDtypeStruct((250000, 128), jnp.float32),
    scratch_shapes=[pltpu.VMEM((_TRB, DIM), jnp.float32)],
)


def _tc_body(pos_ref, neg_ref, out_ref):
    pos_s = jnp.sqrt(pos_ref[...])
    neg_s = jnp.sqrt(neg_ref[...])
    l_pos = jnp.mean(pos_s + jnp.log(1.0 + jnp.exp(-pos_s)))
    l_neg = jnp.mean(jnp.log(1.0 + jnp.exp(-neg_s)))
    out_ref[0, 0] = l_pos + l_neg


_tc_loss = pl.pallas_call(
    _tc_body,
    out_shape=jax.ShapeDtypeStruct((1, 1), jnp.float32),
    out_specs=pl.BlockSpec(memory_space=pltpu.SMEM),
)


@jax.jit
def kernel(triples, neg_tails, ent_emb, rel_emb):
    h = triples[:, 0].astype(jnp.int32).reshape(BATCH // STREAM, STREAM)
    r = triples[:, 1].astype(jnp.int32).reshape(BATCH // STREAM, STREAM)
    t = triples[:, 2].astype(jnp.int32).reshape(BATCH // STREAM, STREAM)
    neg = neg_tails.astype(jnp.int32).reshape(BATCH * NEG // STREAM, STREAM)
    ent_lin = _tc_relayout(ent_emb.T).reshape(1000000, DIM)
    pos_d2, neg_d2 = _sc_call(h, r, t, neg, ent_lin, rel_emb)
    loss = _tc_loss(pos_d2.reshape(BATCH // STREAM, STREAM),
                    neg_d2.reshape(BATCH * NEG // STREAM, STREAM))
    return loss[0, 0]


# speedup vs baseline: 1.0968x; 1.0968x over previous
import functools

import jax
import jax.numpy as jnp
from jax import lax
from jax.experimental import pallas as pl
from jax.experimental.pallas import tpu as pltpu
from jax.experimental.pallas import tpu_sc as plsc

DIM = 32
LANES = 16
NC = 2
NS = 16
NW = NC * NS

BATCH = 16384
NEG = 64
B_PER_W = BATCH // NW
NEGS_PER_W = B_PER_W * NEG
STREAM = 128
CHUNK = 512
N_CHUNK = NEGS_PER_W // CHUNK
GROUPS = CHUNK // LANES
IDX_ROWS_PER_W = NEGS_PER_W // STREAM
HIDX_ROWS = B_PER_W // STREAM
N_ENT_ROWS = 1000000


_TPITCH = LANES + 1


def _lane_sums(ps, tbuf):
    iota = lax.iota(jnp.int32, LANES)
    iota_p = iota * _TPITCH
    for j, p in enumerate(ps):
        plsc.store_scatter(tbuf, [iota_p + j], p)
    vs = [plsc.load_gather(tbuf, [iota + (_TPITCH * l)])
          for l in range(LANES)]
    while len(vs) > 1:
        vs = [vs[i] + vs[i + 1] for i in range(0, len(vs), 2)]
    return vs[0]


def _sc_body(h_hbm, r_hbm, t_hbm, neg_hbm, ent_flat_hbm, rel_hbm,
             pos_out, neg_out,
             hidx_v, ridx_v, tidx_v, negidx_v, buf_hr, buf_rows,
             buf_rows1, posstage, outstage, outstage1, tbuf, tbuf1,
             sem, semg0, semg1, semo0, semo1):
    ent_hbm = ent_flat_hbm
    wid = lax.axis_index("s") * NC + lax.axis_index("c")
    iota = lax.iota(jnp.int32, LANES)

    pltpu.sync_copy(h_hbm.at[pl.ds(wid * HIDX_ROWS, HIDX_ROWS)], hidx_v)
    pltpu.sync_copy(r_hbm.at[pl.ds(wid * HIDX_ROWS, HIDX_ROWS)], ridx_v)
    pltpu.sync_copy(t_hbm.at[pl.ds(wid * HIDX_ROWS, HIDX_ROWS)], tidx_v)
    pltpu.sync_copy(neg_hbm.at[pl.ds(wid * IDX_ROWS_PER_W, IDX_ROWS_PER_W)],
                    negidx_v)

    cps = [pltpu.async_copy(ent_hbm.at[hidx_v.at[j]],
                            buf_hr.at[pl.ds(j * STREAM, STREAM)], sem)
           for j in range(HIDX_ROWS)]
    for cp in cps:
        cp.wait()
    cps = [pltpu.async_copy(rel_hbm.at[ridx_v.at[j]],
                            buf_rows.at[pl.ds(j * STREAM, STREAM)], sem)
           for j in range(HIDX_ROWS)]
    for cp in cps:
        cp.wait()

    def hr_body(i, _):
        buf_hr[i, pl.ds(0, LANES)] = (buf_hr[i, pl.ds(0, LANES)]
                                      + buf_rows[i, pl.ds(0, LANES)])
        buf_hr[i, pl.ds(LANES, LANES)] = (buf_hr[i, pl.ds(LANES, LANES)]
                                          + buf_rows[i, pl.ds(LANES, LANES)])
        return 0
    lax.fori_loop(0, B_PER_W, hr_body, 0)

    cps = [pltpu.async_copy(ent_hbm.at[tidx_v.at[j]],
                            buf_rows.at[pl.ds(j * STREAM, STREAM)], sem)
           for j in range(HIDX_ROWS)]
    for cp in cps:
        cp.wait()

    def pos_body(g, _):
        base = g * LANES
        ps = []
        for j in range(LANES):
            row = base + j
            d0 = buf_hr[row, pl.ds(0, LANES)] - buf_rows[row, pl.ds(0, LANES)]
            d1 = (buf_hr[row, pl.ds(LANES, LANES)]
                  - buf_rows[row, pl.ds(LANES, LANES)])
            ps.append(d0 * d0 + d1 * d1)
        posstage[pl.ds(base, LANES)] = _lane_sums(ps, tbuf)
        return 0
    lax.fori_loop(0, B_PER_W // LANES, pos_body, 0)
    pltpu.sync_copy(posstage, pos_out.at[pl.ds(wid * B_PER_W, B_PER_W)])

    def fire(c, buf, semg):
        for j in range(4):
            pltpu.async_copy(ent_hbm.at[negidx_v.at[c * 4 + j]],
                             buf.at[pl.ds(j * STREAM, STREAM)], semg)

    def drain(buf, semg):
        for j in range(4):
            pltpu.make_async_copy(ent_hbm.at[negidx_v.at[j]],
                                  buf.at[pl.ds(j * STREAM, STREAM)],
                                  semg).wait()

    def fire_out(c, ostage, semo):
        pltpu.async_copy(ostage,
                         neg_out.at[pl.ds(wid * NEGS_PER_W + c * CHUNK,
                                          CHUNK)], semo)

    def drain_out(ostage, semo):
        pltpu.make_async_copy(ostage, neg_out.at[pl.ds(wid * NEGS_PER_W,
                                                       CHUNK)], semo).wait()

    def compute(c, buf, ostage):
        def row_body(b, _):
            local = c * 8 + b
            hr0 = buf_hr[local, pl.ds(0, LANES)]
            hr1 = buf_hr[local, pl.ds(LANES, LANES)]
            for g in range(NEG // LANES):
                base = b * NEG + g * LANES
                ps = []
                for j in range(LANES):
                    row = base + j
                    x0 = buf[row, pl.ds(0, LANES)]
                    x1 = buf[row, pl.ds(LANES, LANES)]
                    d0 = x0 - hr0
                    d1 = x1 - hr1
                    ps.append(d0 * d0 + d1 * d1)
                ostage[pl.ds(base, LANES)] = _lane_sums(
                    ps, tbuf if g % 2 == 0 else tbuf1)
            return 0
        lax.fori_loop(0, CHUNK // NEG, row_body, 0)

    fire(0, buf_rows, semg0)

    def pair_body(p, _):
        c0 = 2 * p
        fire(c0 + 1, buf_rows1, semg1)
        drain(buf_rows, semg0)

        @pl.when(p > 0)
        def _():
            drain_out(outstage, semo0)
        compute(c0, buf_rows, outstage)
        fire_out(c0, outstage, semo0)

        @pl.when(p + 1 < N_CHUNK // 2)
        def _():
            fire(c0 + 2, buf_rows, semg0)
        drain(buf_rows1, semg1)

        @pl.when(p > 0)
        def _():
            drain_out(outstage1, semo1)
        compute(c0 + 1, buf_rows1, outstage1)
        fire_out(c0 + 1, outstage1, semo1)
        return 0
    lax.fori_loop(0, N_CHUNK // 2, pair_body, 0)
    drain_out(outstage, semo0)
    drain_out(outstage1, semo1)


_sc_call = pl.kernel(
    _sc_body,
    out_type=[jax.ShapeDtypeStruct((BATCH,), jnp.float32),
              jax.ShapeDtypeStruct((BATCH * NEG,), jnp.float32)],
    mesh=plsc.VectorSubcoreMesh(core_axis_name="c", subcore_axis_name="s",
                                num_cores=NC, num_subcores=NS),
    compiler_params=pltpu.CompilerParams(needs_layout_passes=False,
                                         use_tc_tiling_on_sc=False),
    scratch_types=[
        pltpu.VMEM((HIDX_ROWS, STREAM), jnp.int32),
        pltpu.VMEM((HIDX_ROWS, STREAM), jnp.int32),
        pltpu.VMEM((HIDX_ROWS, STREAM), jnp.int32),
        pltpu.VMEM((IDX_ROWS_PER_W, STREAM), jnp.int32),
        pltpu.VMEM((CHUNK, DIM), jnp.float32),
        pltpu.VMEM((CHUNK, DIM), jnp.float32),
        pltpu.VMEM((CHUNK, DIM), jnp.float32),
        pltpu.VMEM((B_PER_W,), jnp.float32),
        pltpu.VMEM((CHUNK,), jnp.float32),
        pltpu.VMEM((CHUNK,), jnp.float32),
        pltpu.VMEM((LANES * _TPITCH,), jnp.float32),
        pltpu.VMEM((LANES * _TPITCH,), jnp.float32),
        pltpu.SemaphoreType.DMA,
        pltpu.SemaphoreType.DMA,
        pltpu.SemaphoreType.DMA,
        pltpu.SemaphoreType.DMA,
        pltpu.SemaphoreType.DMA,
    ],
)


_TRB = 8192


def _tc_relayout_body(src_ref, dst_ref):
    eye = jnp.eye(DIM, dtype=jnp.float32)
    dst_ref[:, 0:DIM] = lax.dot_general(
        src_ref[...], eye, (((0,), (0,)), ((), ())),
        preferred_element_type=jnp.float32)


_tc_relayout = pl.pallas_call(
    _tc_relayout_body,
    grid=(123,),
    in_specs=[pl.BlockSpec((DIM, _TRB), lambda j: (0, j))],
    out_specs=pl.BlockSpec((_TRB, 128), lambda j: (j, 0)),
    out_shape=jax.ShapeDtypeStruct((1000000, 128), jnp.float32),
)


def _tc_body(pos_ref, neg_ref, out_ref):
    pos_s = jnp.sqrt(pos_ref[...])
    neg_s = jnp.sqrt(neg_ref[...])
    l_pos = jnp.mean(pos_s + jnp.log(1.0 + jnp.exp(-pos_s)))
    l_neg = jnp.mean(jnp.log(1.0 + jnp.exp(-neg_s)))
    out_ref[0, 0] = l_pos + l_neg


_tc_loss = pl.pallas_call(
    _tc_body,
    out_shape=jax.ShapeDtypeStruct((1, 1), jnp.float32),
    out_specs=pl.BlockSpec(memory_space=pltpu.SMEM),
)


@jax.jit
def kernel(triples, neg_tails, ent_emb, rel_emb):
    h = (triples[:, 0].astype(jnp.int32) * 4).reshape(BATCH // STREAM, STREAM)
    r = triples[:, 1].astype(jnp.int32).reshape(BATCH // STREAM, STREAM)
    t = (triples[:, 2].astype(jnp.int32) * 4).reshape(BATCH // STREAM, STREAM)
    neg = (neg_tails.astype(jnp.int32) * 4).reshape(BATCH * NEG // STREAM,
                                                    STREAM)
    ent_lin = _tc_relayout(ent_emb.T).reshape(4 * 1000000, DIM)
    pos_d2, neg_d2 = _sc_call(h, r, t, neg, ent_lin, rel_emb)
    loss = _tc_loss(pos_d2.reshape(BATCH // STREAM, STREAM),
                    neg_d2.reshape(BATCH * NEG // STREAM, STREAM))
    return loss[0, 0]


# speedup vs baseline: 1.2095x; 1.1028x over previous
import functools

import jax
import jax.numpy as jnp
from jax import lax
from jax.experimental import pallas as pl
from jax.experimental.pallas import tpu as pltpu
from jax.experimental.pallas import tpu_sc as plsc

DIM = 32
LANES = 16
NC = 2
NS = 16
NW = NC * NS

BATCH = 16384
NEG = 64
B_PER_W = BATCH // NW
NEGS_PER_W = B_PER_W * NEG
STREAM = 128
CHUNK = 512
N_CHUNK = NEGS_PER_W // CHUNK
GROUPS = CHUNK // LANES
IDX_ROWS_PER_W = NEGS_PER_W // STREAM
HIDX_ROWS = B_PER_W // STREAM
N_ENT_ROWS = 1000000


_TPITCH = LANES + 1


def _lane_sums(ps, tbuf):
    iota = lax.iota(jnp.int32, LANES)
    iota_p = iota * _TPITCH
    for j, p in enumerate(ps):
        plsc.store_scatter(tbuf, [iota_p + j], p)
    vs = [plsc.load_gather(tbuf, [iota + (_TPITCH * l)])
          for l in range(LANES)]
    while len(vs) > 1:
        vs = [vs[i] + vs[i + 1] for i in range(0, len(vs), 2)]
    return vs[0]


def _sc_body(h_hbm, r_hbm, t_hbm, neg_hbm, ent_flat_hbm, rel_hbm,
             pos_out, neg_out,
             hidx_v, ridx_v, tidx_v, negidx_v, buf_hr, buf_rows,
             buf_rows1, posstage, outstage, outstage1, tbuf, tbuf1,
             sem, semg0, semg1, semo0, semo1):
    ent_hbm = ent_flat_hbm
    wid = lax.axis_index("s") * NC + lax.axis_index("c")
    iota = lax.iota(jnp.int32, LANES)

    pltpu.sync_copy(h_hbm.at[pl.ds(wid * HIDX_ROWS, HIDX_ROWS)], hidx_v)
    pltpu.sync_copy(r_hbm.at[pl.ds(wid * HIDX_ROWS, HIDX_ROWS)], ridx_v)
    pltpu.sync_copy(t_hbm.at[pl.ds(wid * HIDX_ROWS, HIDX_ROWS)], tidx_v)
    pltpu.sync_copy(neg_hbm.at[pl.ds(wid * IDX_ROWS_PER_W, IDX_ROWS_PER_W)],
                    negidx_v)

    cps = [pltpu.async_copy(ent_hbm.at[hidx_v.at[j]],
                            buf_hr.at[pl.ds(j * STREAM, STREAM)], sem)
           for j in range(HIDX_ROWS)]
    for cp in cps:
        cp.wait()
    cps = [pltpu.async_copy(rel_hbm.at[ridx_v.at[j]],
                            buf_rows.at[pl.ds(j * STREAM, STREAM)], sem)
           for j in range(HIDX_ROWS)]
    for cp in cps:
        cp.wait()

    def hr_body(i, _):
        buf_hr[i, pl.ds(0, LANES)] = (buf_hr[i, pl.ds(0, LANES)]
                                      + buf_rows[i, pl.ds(0, LANES)])
        buf_hr[i, pl.ds(LANES, LANES)] = (buf_hr[i, pl.ds(LANES, LANES)]
                                          + buf_rows[i, pl.ds(LANES, LANES)])
        return 0
    lax.fori_loop(0, B_PER_W, hr_body, 0)

    cps = [pltpu.async_copy(ent_hbm.at[tidx_v.at[j]],
                            buf_rows.at[pl.ds(j * STREAM, STREAM)], sem)
           for j in range(HIDX_ROWS)]
    for cp in cps:
        cp.wait()

    def pos_body(g, _):
        base = g * LANES
        ps = []
        for j in range(LANES):
            row = base + j
            d0 = buf_hr[row, pl.ds(0, LANES)] - buf_rows[row, pl.ds(0, LANES)]
            d1 = (buf_hr[row, pl.ds(LANES, LANES)]
                  - buf_rows[row, pl.ds(LANES, LANES)])
            ps.append(d0 * d0 + d1 * d1)
        posstage[pl.ds(base, LANES)] = _lane_sums(ps, tbuf)
        return 0
    lax.fori_loop(0, B_PER_W // LANES, pos_body, 0)
    pltpu.sync_copy(posstage, pos_out.at[pl.ds(wid * B_PER_W, B_PER_W)])

    def fire(c, buf, semg):
        for j in range(4):
            pltpu.async_copy(ent_hbm.at[negidx_v.at[c * 4 + j]],
                             buf.at[pl.ds(j * STREAM, STREAM)], semg)

    def drain(buf, semg):
        for j in range(4):
            pltpu.make_async_copy(ent_hbm.at[negidx_v.at[j]],
                                  buf.at[pl.ds(j * STREAM, STREAM)],
                                  semg).wait()

    def fire_out(c, ostage, semo):
        pltpu.async_copy(ostage,
                         neg_out.at[pl.ds(wid * NEGS_PER_W + c * CHUNK,
                                          CHUNK)], semo)

    def drain_out(ostage, semo):
        pltpu.make_async_copy(ostage, neg_out.at[pl.ds(wid * NEGS_PER_W,
                                                       CHUNK)], semo).wait()

    def compute(c, buf, ostage):
        def row_body(b, _):
            local = c * 8 + b
            hr0 = buf_hr[local, pl.ds(0, LANES)]
            hr1 = buf_hr[local, pl.ds(LANES, LANES)]
            for g in range(NEG // LANES):
                base = b * NEG + g * LANES
                ps = []
                for j in range(LANES):
                    row = base + j
                    x0 = buf[row, pl.ds(0, LANES)]
                    x1 = buf[row, pl.ds(LANES, LANES)]
                    d0 = x0 - hr0
                    d1 = x1 - hr1
                    ps.append(d0 * d0 + d1 * d1)
                ostage[pl.ds(base, LANES)] = _lane_sums(
                    ps, tbuf if g % 2 == 0 else tbuf1)
            return 0
        lax.fori_loop(0, CHUNK // NEG, row_body, 0)

    fire(0, buf_rows, semg0)

    def pair_body(p, _):
        c0 = 2 * p
        fire(c0 + 1, buf_rows1, semg1)
        drain(buf_rows, semg0)

        @pl.when(p > 0)
        def _():
            drain_out(outstage, semo0)
        compute(c0, buf_rows, outstage)
        fire_out(c0, outstage, semo0)

        @pl.when(p + 1 < N_CHUNK // 2)
        def _():
            fire(c0 + 2, buf_rows, semg0)
        drain(buf_rows1, semg1)

        @pl.when(p > 0)
        def _():
            drain_out(outstage1, semo1)
        compute(c0 + 1, buf_rows1, outstage1)
        fire_out(c0 + 1, outstage1, semo1)
        return 0
    lax.fori_loop(0, N_CHUNK // 2, pair_body, 0)
    drain_out(outstage, semo0)
    drain_out(outstage1, semo1)


_sc_call = pl.kernel(
    _sc_body,
    out_type=[jax.ShapeDtypeStruct((BATCH,), jnp.float32),
              jax.ShapeDtypeStruct((BATCH * NEG,), jnp.float32)],
    mesh=plsc.VectorSubcoreMesh(core_axis_name="c", subcore_axis_name="s",
                                num_cores=NC, num_subcores=NS),
    compiler_params=pltpu.CompilerParams(needs_layout_passes=False,
                                         use_tc_tiling_on_sc=False),
    scratch_types=[
        pltpu.VMEM((HIDX_ROWS, STREAM), jnp.int32),
        pltpu.VMEM((HIDX_ROWS, STREAM), jnp.int32),
        pltpu.VMEM((HIDX_ROWS, STREAM), jnp.int32),
        pltpu.VMEM((IDX_ROWS_PER_W, STREAM), jnp.int32),
        pltpu.VMEM((CHUNK, DIM), jnp.float32),
        pltpu.VMEM((CHUNK, DIM), jnp.float32),
        pltpu.VMEM((CHUNK, DIM), jnp.float32),
        pltpu.VMEM((B_PER_W,), jnp.float32),
        pltpu.VMEM((CHUNK,), jnp.float32),
        pltpu.VMEM((CHUNK,), jnp.float32),
        pltpu.VMEM((LANES * _TPITCH,), jnp.float32),
        pltpu.VMEM((LANES * _TPITCH,), jnp.float32),
        pltpu.SemaphoreType.DMA,
        pltpu.SemaphoreType.DMA,
        pltpu.SemaphoreType.DMA,
        pltpu.SemaphoreType.DMA,
        pltpu.SemaphoreType.DMA,
    ],
)


_TRB = 16384


def _tc_relayout_body(src_ref, dst_ref):
    dst_ref[:, 0:DIM] = src_ref[...].T


_tc_relayout = pl.pallas_call(
    _tc_relayout_body,
    grid=(62,),
    in_specs=[pl.BlockSpec((DIM, _TRB), lambda j: (0, j))],
    out_specs=pl.BlockSpec((_TRB, 128), lambda j: (j, 0)),
    out_shape=jax.ShapeDtypeStruct((1000000, 128), jnp.float32),
)


def _tc_body(pos_ref, neg_ref, out_ref):
    pos_s = jnp.sqrt(pos_ref[...])
    neg_s = jnp.sqrt(neg_ref[...])
    l_pos = jnp.mean(pos_s + jnp.log(1.0 + jnp.exp(-pos_s)))
    l_neg = jnp.mean(jnp.log(1.0 + jnp.exp(-neg_s)))
    out_ref[0, 0] = l_pos + l_neg


_tc_loss = pl.pallas_call(
    _tc_body,
    out_shape=jax.ShapeDtypeStruct((1, 1), jnp.float32),
    out_specs=pl.BlockSpec(memory_space=pltpu.SMEM),
)


@jax.jit
def kernel(triples, neg_tails, ent_emb, rel_emb):
    h = (triples[:, 0].astype(jnp.int32) * 4).reshape(BATCH // STREAM, STREAM)
    r = triples[:, 1].astype(jnp.int32).reshape(BATCH // STREAM, STREAM)
    t = (triples[:, 2].astype(jnp.int32) * 4).reshape(BATCH // STREAM, STREAM)
    neg = (neg_tails.astype(jnp.int32) * 4).reshape(BATCH * NEG // STREAM,
                                                    STREAM)
    ent_lin = _tc_relayout(ent_emb.T).reshape(4 * 1000000, DIM)
    pos_d2, neg_d2 = _sc_call(h, r, t, neg, ent_lin, rel_emb)
    loss = _tc_loss(pos_d2.reshape(BATCH // STREAM, STREAM),
                    neg_d2.reshape(BATCH * NEG // STREAM, STREAM))
    return loss[0, 0]


# speedup vs baseline: 1.2291x; 1.0162x over previous
import functools

import jax
import jax.numpy as jnp
from jax import lax
from jax.experimental import pallas as pl
from jax.experimental.pallas import tpu as pltpu
from jax.experimental.pallas import tpu_sc as plsc

DIM = 32
LANES = 16
NC = 2
NS = 16
NW = NC * NS

BATCH = 16384
NEG = 64
B_PER_W = BATCH // NW
NEGS_PER_W = B_PER_W * NEG
STREAM = 128
CHUNK = 512
N_CHUNK = NEGS_PER_W // CHUNK
GROUPS = CHUNK // LANES
IDX_ROWS_PER_W = NEGS_PER_W // STREAM
HIDX_ROWS = B_PER_W // STREAM
N_ENT_ROWS = 1000000


_TPITCH = LANES + 1


def _lane_sums(ps, tbuf):
    iota = lax.iota(jnp.int32, LANES)
    iota_p = iota * _TPITCH
    for j, p in enumerate(ps):
        plsc.store_scatter(tbuf, [iota_p + j], p)
    vs = [plsc.load_gather(tbuf, [iota + (_TPITCH * l)])
          for l in range(LANES)]
    while len(vs) > 1:
        vs = [vs[i] + vs[i + 1] for i in range(0, len(vs), 2)]
    return vs[0]


def _sc_body(h_hbm, r_hbm, t_hbm, neg_hbm, ent_flat_hbm, rel_hbm,
             pos_out, neg_out,
             hidx_v, ridx_v, tidx_v, negidx_v, buf_hr, buf_rows,
             buf_rows1, posstage, outstage, outstage1, tbuf, tbuf1,
             sem, semg0, semg1, semo0, semo1):
    ent_hbm = ent_flat_hbm
    wid = lax.axis_index("s") * NC + lax.axis_index("c")
    iota = lax.iota(jnp.int32, LANES)

    pltpu.sync_copy(h_hbm.at[pl.ds(wid * HIDX_ROWS, HIDX_ROWS)], hidx_v)
    pltpu.sync_copy(r_hbm.at[pl.ds(wid * HIDX_ROWS, HIDX_ROWS)], ridx_v)
    pltpu.sync_copy(t_hbm.at[pl.ds(wid * HIDX_ROWS, HIDX_ROWS)], tidx_v)
    pltpu.sync_copy(neg_hbm.at[pl.ds(wid * IDX_ROWS_PER_W, IDX_ROWS_PER_W)],
                    negidx_v)

    cps = [pltpu.async_copy(ent_hbm.at[hidx_v.at[j]],
                            buf_hr.at[pl.ds(j * STREAM, STREAM)], sem)
           for j in range(HIDX_ROWS)]
    for cp in cps:
        cp.wait()
    cps = [pltpu.async_copy(rel_hbm.at[ridx_v.at[j]],
                            buf_rows.at[pl.ds(j * STREAM, STREAM)], sem)
           for j in range(HIDX_ROWS)]
    for cp in cps:
        cp.wait()

    def hr_body(i, _):
        buf_hr[i, pl.ds(0, LANES)] = (buf_hr[i, pl.ds(0, LANES)]
                                      + buf_rows[i, pl.ds(0, LANES)])
        buf_hr[i, pl.ds(LANES, LANES)] = (buf_hr[i, pl.ds(LANES, LANES)]
                                          + buf_rows[i, pl.ds(LANES, LANES)])
        return 0
    lax.fori_loop(0, B_PER_W, hr_body, 0)

    cps = [pltpu.async_copy(ent_hbm.at[tidx_v.at[j]],
                            buf_rows.at[pl.ds(j * STREAM, STREAM)], sem)
           for j in range(HIDX_ROWS)]
    for cp in cps:
        cp.wait()

    def pos_body(g, _):
        base = g * LANES
        ps = []
        for j in range(LANES):
            row = base + j
            d0 = buf_hr[row, pl.ds(0, LANES)] - buf_rows[row, pl.ds(0, LANES)]
            d1 = (buf_hr[row, pl.ds(LANES, LANES)]
                  - buf_rows[row, pl.ds(LANES, LANES)])
            ps.append(d0 * d0 + d1 * d1)
        posstage[pl.ds(base, LANES)] = _lane_sums(ps, tbuf)
        return 0
    lax.fori_loop(0, B_PER_W // LANES, pos_body, 0)
    pltpu.sync_copy(posstage, pos_out.at[pl.ds(wid * B_PER_W, B_PER_W)])

    def fire(c, buf, semg):
        for j in range(4):
            pltpu.async_copy(ent_hbm.at[negidx_v.at[c * 4 + j]],
                             buf.at[pl.ds(j * STREAM, STREAM)], semg)

    def drain(buf, semg):
        for j in range(4):
            pltpu.make_async_copy(ent_hbm.at[negidx_v.at[j]],
                                  buf.at[pl.ds(j * STREAM, STREAM)],
                                  semg).wait()

    def fire_out(c, ostage, semo):
        pltpu.async_copy(ostage,
                         neg_out.at[pl.ds(wid * NEGS_PER_W + c * CHUNK,
                                          CHUNK)], semo)

    def drain_out(ostage, semo):
        pltpu.make_async_copy(ostage, neg_out.at[pl.ds(wid * NEGS_PER_W,
                                                       CHUNK)], semo).wait()

    def compute(c, buf, ostage):
        def row_body(b, _):
            local = c * 8 + b
            hr0 = buf_hr[local, pl.ds(0, LANES)]
            hr1 = buf_hr[local, pl.ds(LANES, LANES)]
            for g in range(NEG // LANES):
                base = b * NEG + g * LANES
                ps = []
                for j in range(LANES):
                    row = base + j
                    x0 = buf[row, pl.ds(0, LANES)]
                    x1 = buf[row, pl.ds(LANES, LANES)]
                    d0 = x0 - hr0
                    d1 = x1 - hr1
                    ps.append(d0 * d0 + d1 * d1)
                ostage[pl.ds(base, LANES)] = _lane_sums(
                    ps, tbuf if g % 2 == 0 else tbuf1)
            return 0
        lax.fori_loop(0, CHUNK // NEG, row_body, 0)

    fire(0, buf_rows, semg0)

    def pair_body(p, _):
        c0 = 2 * p
        fire(c0 + 1, buf_rows1, semg1)
        drain(buf_rows, semg0)

        @pl.when(p > 0)
        def _():
            drain_out(outstage, semo0)
        compute(c0, buf_rows, outstage)
        fire_out(c0, outstage, semo0)

        @pl.when(p + 1 < N_CHUNK // 2)
        def _():
            fire(c0 + 2, buf_rows, semg0)
        drain(buf_rows1, semg1)

        @pl.when(p > 0)
        def _():
            drain_out(outstage1, semo1)
        compute(c0 + 1, buf_rows1, outstage1)
        fire_out(c0 + 1, outstage1, semo1)
        return 0
    lax.fori_loop(0, N_CHUNK // 2, pair_body, 0)
    drain_out(outstage, semo0)
    drain_out(outstage1, semo1)


_sc_call = pl.kernel(
    _sc_body,
    out_type=[jax.ShapeDtypeStruct((BATCH,), jnp.float32),
              jax.ShapeDtypeStruct((BATCH * NEG,), jnp.float32)],
    mesh=plsc.VectorSubcoreMesh(core_axis_name="c", subcore_axis_name="s",
                                num_cores=NC, num_subcores=NS),
    compiler_params=pltpu.CompilerParams(needs_layout_passes=False,
                                         use_tc_tiling_on_sc=False),
    scratch_types=[
        pltpu.VMEM((HIDX_ROWS, STREAM), jnp.int32),
        pltpu.VMEM((HIDX_ROWS, STREAM), jnp.int32),
        pltpu.VMEM((HIDX_ROWS, STREAM), jnp.int32),
        pltpu.VMEM((IDX_ROWS_PER_W, STREAM), jnp.int32),
        pltpu.VMEM((CHUNK, DIM), jnp.float32),
        pltpu.VMEM((CHUNK, DIM), jnp.float32),
        pltpu.VMEM((CHUNK, DIM), jnp.float32),
        pltpu.VMEM((B_PER_W,), jnp.float32),
        pltpu.VMEM((CHUNK,), jnp.float32),
        pltpu.VMEM((CHUNK,), jnp.float32),
        pltpu.VMEM((LANES * _TPITCH,), jnp.float32),
        pltpu.VMEM((LANES * _TPITCH,), jnp.float32),
        pltpu.SemaphoreType.DMA,
        pltpu.SemaphoreType.DMA,
        pltpu.SemaphoreType.DMA,
        pltpu.SemaphoreType.DMA,
        pltpu.SemaphoreType.DMA,
    ],
)


_TRB = 32768


def _tc_relayout_body(src_ref, dst_ref):
    dst_ref[:, 0:DIM] = src_ref[...].T


_tc_relayout = pl.pallas_call(
    _tc_relayout_body,
    grid=(31,),
    in_specs=[pl.BlockSpec((DIM, _TRB), lambda j: (0, j))],
    out_specs=pl.BlockSpec((_TRB, 128), lambda j: (j, 0)),
    out_shape=jax.ShapeDtypeStruct((1000000, 128), jnp.float32),
)


def _tc_body(pos_ref, neg_ref, out_ref):
    pos_s = jnp.sqrt(pos_ref[...])
    neg_s = jnp.sqrt(neg_ref[...])
    l_pos = jnp.mean(pos_s + jnp.log(1.0 + jnp.exp(-pos_s)))
    l_neg = jnp.mean(jnp.log(1.0 + jnp.exp(-neg_s)))
    out_ref[0, 0] = l_pos + l_neg


_tc_loss = pl.pallas_call(
    _tc_body,
    out_shape=jax.ShapeDtypeStruct((1, 1), jnp.float32),
    out_specs=pl.BlockSpec(memory_space=pltpu.SMEM),
)


@jax.jit
def kernel(triples, neg_tails, ent_emb, rel_emb):
    h = (triples[:, 0].astype(jnp.int32) * 4).reshape(BATCH // STREAM, STREAM)
    r = triples[:, 1].astype(jnp.int32).reshape(BATCH // STREAM, STREAM)
    t = (triples[:, 2].astype(jnp.int32) * 4).reshape(BATCH // STREAM, STREAM)
    neg = (neg_tails.astype(jnp.int32) * 4).reshape(BATCH * NEG // STREAM,
                                                    STREAM)
    ent_lin = _tc_relayout(ent_emb.T).reshape(4 * 1000000, DIM)
    pos_d2, neg_d2 = _sc_call(h, r, t, neg, ent_lin, rel_emb)
    loss = _tc_loss(pos_d2.reshape(BATCH // STREAM, STREAM),
                    neg_d2.reshape(BATCH * NEG // STREAM, STREAM))
    return loss[0, 0]
